# Initial kernel scaffold; baseline (speedup 1.0000x reference)
import jax
import jax.numpy as jnp
from jax.experimental import pallas as pl


def kernel(xh, node_mask, edge_mask, context, params):
    raise NotImplementedError("write your pallas kernel here")


# speedup vs baseline: 5.1818x; 5.1818x over previous
import jax
import jax.numpy as jnp
from jax.experimental import pallas as pl
from jax.experimental.pallas import tpu as pltpu

BS = 16
N = 48
P = N * N
H = 128
NDIM = 3
XL = 8
N_LAYERS = 4
INV_SUBLAYERS = 2
INV_NORM = 1.0 / 100.0


def _dot(a, b):
    return jax.lax.dot_general(
        a, b, (((1,), (0,)), ((), ())),
        precision=jax.lax.Precision.HIGHEST,
        preferred_element_type=jnp.float32)


def _rep_i(v):
    return jnp.broadcast_to(v[:, None, :], (N, N, v.shape[-1])).reshape(P, -1)


def _rep_j(v):
    return jnp.broadcast_to(v[None, :, :], (N, N, v.shape[-1])).reshape(P, -1)


def _seg_sum(e):
    return e.reshape(N, N, e.shape[-1]).sum(axis=1)


def _egnn_body(x_ref, h0_ref, nm_ref, em_ref,
               wemb_ref, bemb_ref, gm_ref, gv_ref, cm_ref, cv_ref,
               wout_ref, bout_ref,
               vel_ref, hf_ref):
    nm = nm_ref[0]
    em = em_ref[0]
    x = x_ref[0]
    h = _dot(h0_ref[0], wemb_ref[...]) + bemb_ref[...]

    cd0 = _rep_i(x) - _rep_j(x)
    d0 = jnp.sum(cd0 * cd0, axis=1, keepdims=True)

    for blk in range(N_LAYERS):
        cd = _rep_i(x) - _rep_j(x)
        r = jnp.sum(cd * cd, axis=1, keepdims=True)
        cdn = cd / (jnp.sqrt(r + 1e-8) + 1.0)

        for s in range(INV_SUBLAYERS):
            g = blk * INV_SUBLAYERS + s
            a = _dot(h, gm_ref[g, 0])
            b = _dot(h, gm_ref[g, 1])
            pre = (_rep_i(a) + _rep_j(b)
                   + r * gv_ref[g, 0] + d0 * gv_ref[g, 1] + gv_ref[g, 2])
            mij = jax.nn.silu(_dot(jax.nn.silu(pre), gm_ref[g, 2])
                              + gv_ref[g, 3])
            att = jax.nn.sigmoid(
                jnp.sum(mij * gv_ref[g, 6], axis=1, keepdims=True)
                + gv_ref[g, 7:8, 0:1])
            agg = _seg_sum(mij * att * em) * INV_NORM
            t = jax.nn.silu(_dot(h, gm_ref[g, 3]) + _dot(agg, gm_ref[g, 4])
                            + gv_ref[g, 4])
            h = (h + _dot(t, gm_ref[g, 5]) + gv_ref[g, 5]) * nm

        c = _dot(h, cm_ref[blk, 0])
        d = _dot(h, cm_ref[blk, 1])
        pre = (_rep_i(c) + _rep_j(d)
               + r * cv_ref[blk, 0] + d0 * cv_ref[blk, 1] + cv_ref[blk, 2])
        t = jax.nn.silu(_dot(jax.nn.silu(pre), cm_ref[blk, 2])
                        + cv_ref[blk, 3])
        phi = jnp.sum(t * cv_ref[blk, 4], axis=1, keepdims=True)
        aggx = _seg_sum(cdn * phi * em) * INV_NORM
        x = (x + aggx) * nm
        h = h * nm

    hf = (_dot(h, wout_ref[...]) + bout_ref[...]) * nm
    vel = x * nm
    ncnt = jnp.sum(nm)
    mean = jnp.sum(vel, axis=0, keepdims=True) / ncnt
    vel_ref[0] = vel - mean * nm
    hf_ref[0] = hf


def kernel(xh, node_mask, edge_mask, context, params):
    nm = node_mask.reshape(BS, N, 1)
    xh = xh.reshape(BS, N, -1) * nm
    x0 = jnp.pad(xh[..., :NDIM], ((0, 0), (0, 0), (0, XL - NDIM)))
    h0 = jnp.concatenate([xh[..., NDIM:], context.reshape(BS, N, -1)], axis=-1)
    h0 = jnp.pad(h0, ((0, 0), (0, 0), (0, 16 - h0.shape[-1])))
    em = edge_mask.reshape(BS, P, 1)

    wemb = jnp.pad(params["emb"]["W"].T, ((0, 6), (0, 0)))
    bemb = params["emb"]["b"].reshape(1, H)
    wout = jnp.pad(params["emb_out"]["W"].T, ((0, 0), (0, 2)))
    bout = jnp.pad(params["emb_out"]["b"], (0, 2)).reshape(1, 8)

    gms, gvs, cms, cvs = [], [], [], []
    for blk in params["blocks"]:
        for gp in blk["gcls"]:
            e0 = gp["edge0"]["W"]
            gms.append(jnp.stack([
                e0[:, :H].T, e0[:, H:2 * H].T, gp["edge1"]["W"].T,
                gp["node0"]["W"][:, :H].T, gp["node0"]["W"][:, H:].T,
                gp["node1"]["W"].T]))
            gvs.append(jnp.stack([
                e0[:, 2 * H], e0[:, 2 * H + 1], gp["edge0"]["b"],
                gp["edge1"]["b"], gp["node0"]["b"], gp["node1"]["b"],
                gp["att"]["W"][0], jnp.full((H,), gp["att"]["b"][0])]))
        c0 = blk["coord0"]["W"]
        cms.append(jnp.stack([
            c0[:, :H].T, c0[:, H:2 * H].T, blk["coord1"]["W"].T]))
        cvs.append(jnp.stack([
            c0[:, 2 * H], c0[:, 2 * H + 1], blk["coord0"]["b"],
            blk["coord1"]["b"], blk["coord2"]["W"][0],
            jnp.zeros((H,)), jnp.zeros((H,)), jnp.zeros((H,))]))
    gm = jnp.stack(gms)
    gv = jnp.stack(gvs)
    cm = jnp.stack(cms)
    cv = jnp.stack(cvs)

    full = lambda a: pl.BlockSpec(a.shape, lambda i: (0,) * a.ndim)
    batched = lambda a: pl.BlockSpec((1,) + a.shape[1:],
                                     lambda i: (i,) + (0,) * (a.ndim - 1))

    vel, hf = pl.pallas_call(
        _egnn_body,
        grid=(BS,),
        in_specs=[batched(x0), batched(h0), batched(nm), batched(em),
                  full(wemb), full(bemb), full(gm), full(gv),
                  full(cm), full(cv), full(wout), full(bout)],
        out_specs=[pl.BlockSpec((1, N, XL), lambda i: (i, 0, 0)),
                   pl.BlockSpec((1, N, 8), lambda i: (i, 0, 0))],
        out_shape=[jax.ShapeDtypeStruct((BS, N, XL), jnp.float32),
                   jax.ShapeDtypeStruct((BS, N, 8), jnp.float32)],
        compiler_params=pltpu.CompilerParams(
            dimension_semantics=("parallel",)),
    )(x0, h0, nm, em, wemb, bemb, gm, gv, cm, cv, wout, bout)

    vel = vel[..., :NDIM]
    vel = jnp.where(jnp.any(jnp.isnan(vel)), jnp.zeros_like(vel), vel)
    return vel, hf[..., :params["emb_out"]["W"].shape[0]]


# speedup vs baseline: 15.9760x; 3.0831x over previous
import jax
import jax.numpy as jnp
from jax.experimental import pallas as pl
from jax.experimental.pallas import tpu as pltpu

BS = 16
N = 48
P = N * N
H = 128
NDIM = 3
XL = 8
N_LAYERS = 4
INV_SUBLAYERS = 2
INV_NORM = 1.0 / 100.0

MB = 2
NM = MB * N
MP = MB * P


def _dot(a, b):
    return jax.lax.dot_general(
        a, b, (((1,), (0,)), ((), ())),
        preferred_element_type=jnp.float32)


def _pair(a, b, op):
    L = a.shape[-1]
    return op(a.reshape(MB, N, 1, L), b.reshape(MB, 1, N, L)).reshape(MP, L)


def _seg_sum(e):
    return e.reshape(NM, N, e.shape[-1]).sum(axis=1)


def _egnn_body(x_ref, h0_ref, nm_ref, em_ref,
               wemb_ref, bemb_ref, gm_ref, gv_ref, cm_ref, cv_ref,
               wout_ref, bout_ref,
               vel_ref, hf_ref):
    nm = nm_ref[0]
    em = em_ref[0]
    x = x_ref[0]
    h = _dot(h0_ref[0], wemb_ref[...]) + bemb_ref[...]

    cd0 = _pair(x, x, jnp.subtract)
    d0 = jnp.sum(cd0 * cd0, axis=1, keepdims=True)

    for blk in range(N_LAYERS):
        cd = _pair(x, x, jnp.subtract)
        r = jnp.sum(cd * cd, axis=1, keepdims=True)
        cdn = cd * (1.0 / (jnp.sqrt(r + 1e-8) + 1.0))
        ea = jnp.concatenate([r, d0], axis=1)

        for s in range(INV_SUBLAYERS):
            g = blk * INV_SUBLAYERS + s
            a = _dot(h, gm_ref[g, 0]) + gv_ref[g, 2]
            b = _dot(h, gm_ref[g, 1])
            pre = _pair(a, b, jnp.add) + _dot(ea, gv_ref[g, 0:2])
            mij = jax.nn.silu(_dot(jax.nn.silu(pre), gm_ref[g, 2])
                              + gv_ref[g, 3])
            att = jax.nn.sigmoid(
                jnp.sum(mij * gv_ref[g, 6], axis=1, keepdims=True)
                + gv_ref[g, 7:8, 0:1]) * em
            agg = _seg_sum(mij * att) * INV_NORM
            t = jax.nn.silu(_dot(h, gm_ref[g, 3]) + _dot(agg, gm_ref[g, 4])
                            + gv_ref[g, 4])
            h = (h + _dot(t, gm_ref[g, 5]) + gv_ref[g, 5]) * nm

        c = _dot(h, cm_ref[blk, 0]) + cv_ref[blk, 2]
        d = _dot(h, cm_ref[blk, 1])
        pre = _pair(c, d, jnp.add) + _dot(ea, cv_ref[blk, 0:2])
        t = jax.nn.silu(_dot(jax.nn.silu(pre), cm_ref[blk, 2])
                        + cv_ref[blk, 3])
        phi = jnp.sum(t * cv_ref[blk, 4], axis=1, keepdims=True) * em
        aggx = _seg_sum(cdn * phi) * INV_NORM
        x = (x + aggx) * nm
        h = h * nm

    hf = (_dot(h, wout_ref[...]) + bout_ref[...]) * nm
    v3 = (x * nm).reshape(MB, N, XL)
    nm3 = nm.reshape(MB, N, 1)
    ncnt = jnp.sum(nm3, axis=1, keepdims=True)
    mean = jnp.sum(v3, axis=1, keepdims=True) / ncnt
    vel_ref[0] = (v3 - mean * nm3).reshape(NM, XL)
    hf_ref[0] = hf


def kernel(xh, node_mask, edge_mask, context, params):
    nm = node_mask.reshape(BS, N, 1)
    xh = xh.reshape(BS, N, -1) * nm
    x0 = jnp.pad(xh[..., :NDIM], ((0, 0), (0, 0), (0, XL - NDIM)))
    h0 = jnp.concatenate([xh[..., NDIM:], context.reshape(BS, N, -1)], axis=-1)
    h0 = jnp.pad(h0, ((0, 0), (0, 0), (0, 16 - h0.shape[-1])))

    grid = BS // MB
    x0 = x0.reshape(grid, NM, XL)
    h0 = h0.reshape(grid, NM, 16)
    nm = nm.reshape(grid, NM, 1)
    em = edge_mask.reshape(grid, MP, 1)

    wemb = jnp.pad(params["emb"]["W"].T, ((0, 6), (0, 0)))
    bemb = params["emb"]["b"].reshape(1, H)
    wout = jnp.pad(params["emb_out"]["W"].T, ((0, 0), (0, 2)))
    bout = jnp.pad(params["emb_out"]["b"], (0, 2)).reshape(1, 8)

    gms, gvs, cms, cvs = [], [], [], []
    for blk in params["blocks"]:
        for gp in blk["gcls"]:
            e0 = gp["edge0"]["W"]
            gms.append(jnp.stack([
                e0[:, :H].T, e0[:, H:2 * H].T, gp["edge1"]["W"].T,
                gp["node0"]["W"][:, :H].T, gp["node0"]["W"][:, H:].T,
                gp["node1"]["W"].T]))
            gvs.append(jnp.stack([
                e0[:, 2 * H], e0[:, 2 * H + 1], gp["edge0"]["b"],
                gp["edge1"]["b"], gp["node0"]["b"], gp["node1"]["b"],
                gp["att"]["W"][0], jnp.full((H,), gp["att"]["b"][0])]))
        c0 = blk["coord0"]["W"]
        cms.append(jnp.stack([
            c0[:, :H].T, c0[:, H:2 * H].T, blk["coord1"]["W"].T]))
        cvs.append(jnp.stack([
            c0[:, 2 * H], c0[:, 2 * H + 1], blk["coord0"]["b"],
            blk["coord1"]["b"], blk["coord2"]["W"][0],
            jnp.zeros((H,)), jnp.zeros((H,)), jnp.zeros((H,))]))
    gm = jnp.stack(gms)
    gv = jnp.stack(gvs)
    cm = jnp.stack(cms)
    cv = jnp.stack(cvs)

    full = lambda a: pl.BlockSpec(a.shape, lambda i: (0,) * a.ndim)
    batched = lambda a: pl.BlockSpec((1,) + a.shape[1:],
                                     lambda i: (i,) + (0,) * (a.ndim - 1))

    vel, hf = pl.pallas_call(
        _egnn_body,
        grid=(grid,),
        in_specs=[batched(x0), batched(h0), batched(nm), batched(em),
                  full(wemb), full(bemb), full(gm), full(gv),
                  full(cm), full(cv), full(wout), full(bout)],
        out_specs=[pl.BlockSpec((1, NM, XL), lambda i: (i, 0, 0)),
                   pl.BlockSpec((1, NM, 8), lambda i: (i, 0, 0))],
        out_shape=[jax.ShapeDtypeStruct((grid, NM, XL), jnp.float32),
                   jax.ShapeDtypeStruct((grid, NM, 8), jnp.float32)],
        compiler_params=pltpu.CompilerParams(
            dimension_semantics=("parallel",)),
    )(x0, h0, nm, em, wemb, bemb, gm, gv, cm, cv, wout, bout)

    vel = vel.reshape(BS, N, XL)[..., :NDIM]
    vel = jnp.where(jnp.any(jnp.isnan(vel)), jnp.zeros_like(vel), vel)
    return vel, hf.reshape(BS, N, 8)[..., :params["emb_out"]["W"].shape[0]]
